# baseline (speedup 1.0000x reference)
import functools

import jax
import jax.numpy as jnp
from jax import lax
from jax.experimental import pallas as pl
from jax.experimental.pallas import tpu as pltpu
from jax.experimental.pallas import tpu_sc as plsc

_B, _S, _H, _V, _P = 64, 512, 1024, 50265, 50
_PAD = 1
_EPS = 1e-5

_NC, _NS = 2, 16
_NW = _NC * _NS
_SWID = _S // _NW
_NRING = 4
_UNROLL = 8

_sc_mesh = plsc.VectorSubcoreMesh(core_axis_name="c", subcore_axis_name="s")


def _xsum(x):
    idx = lax.broadcasted_iota(jnp.int32, (16,), 0)
    dnums = lax.GatherDimensionNumbers(
        offset_dims=(), collapsed_slice_dims=(0,), start_index_map=(0,))
    for sft in (8, 4, 2, 1):
        perm = lax.bitwise_xor(idx, jnp.int32(sft)).reshape(16, 1)
        x = x + lax.gather(x, perm, dnums, (1,),
                           mode=lax.GatherScatterMode.PROMISE_IN_BOUNDS)
    return x


def _vrsqrt(v):
    i = lax.bitcast_convert_type(v, jnp.int32)
    i = jnp.int32(0x5F3759DF) - lax.shift_right_logical(i, 1)
    y = lax.bitcast_convert_type(i, jnp.float32)
    for _ in range(3):
        y = y * (1.5 - 0.5 * v * y * y)
    return y


@functools.partial(
    pl.kernel,
    mesh=_sc_mesh,
    out_type=jax.ShapeDtypeStruct((_B * _S, _H), jnp.float32),
    scratch_types=[
        pltpu.VMEM((_B * _SWID,), jnp.int32),
        pltpu.VMEM((_SWID, _H), jnp.float32),
        pltpu.VMEM((_SWID, _H), jnp.float32),
        pltpu.VMEM((_SWID, _H), jnp.float32),
        pltpu.VMEM((_SWID, _H), jnp.float32),
        pltpu.VMEM((_SWID, _H), jnp.float32),
        pltpu.VMEM((_SWID, _H), jnp.float32),
        pltpu.VMEM((_SWID, _H), jnp.float32),
        pltpu.VMEM((_H,), jnp.float32),
        pltpu.VMEM((_H,), jnp.float32),
        pltpu.SemaphoreType.DMA,
        pltpu.SemaphoreType.DMA,
        pltpu.SemaphoreType.DMA,
        pltpu.SemaphoreType.DMA,
        pltpu.SemaphoreType.DMA,
        pltpu.SemaphoreType.DMA,
    ],
)
def _sc_fused(ids_hbm, table_hbm, padd_hbm, g_hbm, bt_hbm, out_hbm,
              idx_v, x0, x1, x2, x3, pbuf, o0, o1, gbuf, bbuf,
              gs0, gs1, gs2, gs3, ws0, ws1):
    wid = lax.axis_index("s") * _NC + lax.axis_index("c")
    s0 = wid * _SWID

    xb = (x0, x1, x2, x3)
    ob = (o0, o1)
    gs = (gs0, gs1, gs2, gs3)
    ws = (ws0, ws1)

    def gdesc(b, m):
        return pltpu.make_async_copy(
            table_hbm.at[idx_v.at[pl.ds(b * _SWID, _SWID)]], xb[m], gs[m])

    def wdesc(b, pm):
        return pltpu.make_async_copy(
            ob[pm], out_hbm.at[pl.ds(b * _S + s0, _SWID)], ws[pm])

    pltpu.sync_copy(ids_hbm.at[pl.ds(wid * (_B * _SWID), _B * _SWID)], idx_v)
    pltpu.sync_copy(padd_hbm.at[pl.ds(s0, _SWID)], pbuf)
    pltpu.sync_copy(g_hbm, gbuf)
    pltpu.sync_copy(bt_hbm, bbuf)

    for m in range(_NRING):
        gdesc(m, m).start()

    zeros = jnp.zeros((16,), jnp.float32)
    inv_h = jnp.float32(1.0 / _H)
    mf = [jnp.where(s0 + sl >= _P, jnp.float32(1.0), jnp.float32(0.0))
          for sl in range(_SWID)]
    _HALF = _SWID // 2

    def outer(k, carry):
        for m in range(_NRING):
            b = _NRING * k + m
            pm = m % 2
            gdesc(b, m).wait()

            means = []
            rstds = []
            for half in range(2):
                sls = list(range(half * _HALF, (half + 1) * _HALF))

                def pass1(t, c, m=m, sls=sls):
                    cl = list(c)
                    for i, sl in enumerate(sls):
                        for u in range(_UNROLL):
                            off = (_UNROLL * t + u) * 16
                            x = xb[m][sl, pl.ds(off, 16)]
                            p = pbuf[sl, pl.ds(off, 16)]
                            y = x * mf[sl] + p
                            xb[m][sl, pl.ds(off, 16)] = y
                            cl[2 * i] = cl[2 * i] + y
                            cl[2 * i + 1] = cl[2 * i + 1] + y * y
                    return tuple(cl)

                res = lax.fori_loop(0, _H // (16 * _UNROLL), pass1,
                                    (zeros,) * (2 * _HALF))
                for i in range(_HALF):
                    mean = _xsum(res[2 * i]) * inv_h
                    var = _xsum(res[2 * i + 1]) * inv_h - mean * mean
                    means.append(mean)
                    rstds.append(_vrsqrt(var + _EPS))

            if m >= 2:
                wdesc(b, pm).wait()
            else:
                @pl.when(k > 0)
                def _():
                    wdesc(b, pm).wait()

            for half in range(2):
                sls = list(range(half * _HALF, (half + 1) * _HALF))

                def pass2(j, c, m=m, pm=pm, sls=sls):
                    gj = gbuf[pl.ds(16 * j, 16)]
                    bj = bbuf[pl.ds(16 * j, 16)]
                    for sl in sls:
                        y = xb[m][sl, pl.ds(16 * j, 16)]
                        ob[pm][sl, pl.ds(16 * j, 16)] = (
                            (y - means[sl]) * rstds[sl] * gj + bj)
                    return c

                lax.fori_loop(0, _H // 16, pass2, 0)
            wdesc(b, pm).start()

            @pl.when(k < _B // _NRING - 1)
            def _(b=b, m=m):
                gdesc(b + _NRING, m).start()

        return carry

    lax.fori_loop(0, _B // _NRING, outer, 0)
    wdesc(0, 0).wait()
    wdesc(0, 1).wait()


def _mlp_padd_body(p_ref, w1_ref, b1_ref, w2_ref, b2_ref, pos_ref, type_ref,
                   o_ref):
    h = jnp.dot(p_ref[...], w1_ref[...], preferred_element_type=jnp.float32)
    h = jnp.maximum(h + b1_ref[...], 0.0)
    mlp = jnp.dot(h, w2_ref[...], preferred_element_type=jnp.float32) + b2_ref[...]
    r = lax.broadcasted_iota(jnp.int32, (_S, 1), 0)
    o_ref[...] = jnp.where(r < _P, mlp, 0.0) + pos_ref[...] + type_ref[...]


def kernel(input_ids, prompt_pos, word_table, prompt_table, W1, b1, W2, b2,
           pos_table, type_table, ln_gamma, ln_beta):
    ids_w = (input_ids.astype(jnp.int32)
             .reshape(_B, _NW, _SWID)
             .transpose(1, 0, 2)
             .reshape(_B * _S))

    p_pad = jnp.zeros((_S, _H), jnp.float32).at[:_P].set(prompt_table)
    pos_slice = lax.slice(pos_table, (_PAD + 1, 0), (_PAD + 1 + _S, _H))
    padd = pl.pallas_call(
        _mlp_padd_body,
        out_shape=jax.ShapeDtypeStruct((_S, _H), jnp.float32),
    )(p_pad, W1, b1.reshape(1, _H), W2, b2.reshape(1, _H), pos_slice,
      type_table)

    out = _sc_fused(ids_w, word_table, padd, ln_gamma, ln_beta)
    return out.reshape(_B, _S, _H)
